# Initial kernel scaffold; baseline (speedup 1.0000x reference)
import jax
import jax.numpy as jnp
from jax.experimental import pallas as pl


def kernel(x, edge_index, batch, W1, b1, W3, b3, Wg, bg):
    raise NotImplementedError("write your pallas kernel here")


# speedup vs baseline: 6.5539x; 6.5539x over previous
import functools

import jax
import jax.numpy as jnp
from jax import lax
from jax.experimental import pallas as pl
from jax.experimental.pallas import tpu as pltpu
from jax.experimental.pallas import tpu_sc as plsc

N = 100000
E = 1600000
LATENT = 128
DIM_HALF = 32
DIM_IN = 128

NC = 2
NS = 16
NW = NC * NS

EPT = E // NW
HSLICE = 6272
HPAD = NS * HSLICE

R = 10368
PASSES = 5
NPAD = NC * PASSES * R
JROW = R
EPS = E // NS
B_BLK = 10000
NV = B_BLK // 16
CAP = 10144
K = 128
ZROWS = R // NS


def _zero_i32(ref, n):
    def body(i, _):
        ref[pl.ds(i * 16, 16)] = jnp.zeros((16,), jnp.int32)
        return 0
    lax.fori_loop(0, n // 16, body, 0)


def _hist_body(dst_hbm, out_hbm, idx_v, tidx_v, ones_v, zrow_v, shared_hist):
    c = lax.axis_index("c")
    s = lax.axis_index("s")
    wid = c * NS + s

    def fill_ones(i, _):
        ones_v[pl.ds(i * 16, 16)] = jnp.ones((16,), jnp.float32)
        return 0
    lax.fori_loop(0, K // 16, fill_ones, 0)

    def fill_z(i, _):
        zrow_v[pl.ds(i * 16, 16)] = jnp.zeros((16,), jnp.float32)
        return 0
    lax.fori_loop(0, HSLICE // 16, fill_z, 0)

    pltpu.sync_copy(zrow_v, shared_hist.at[pl.ds(s * HSLICE, HSLICE)])
    plsc.subcore_barrier()

    base = wid * EPT
    nfull = EPT // K

    def chunk(j, _):
        pltpu.sync_copy(dst_hbm.at[pl.ds(base + j * K, K)], idx_v)
        pltpu.sync_copy(ones_v, shared_hist.at[idx_v], add=True)
        return 0
    lax.fori_loop(0, nfull, chunk, 0)
    tail = EPT - nfull * K
    if tail:
        pltpu.sync_copy(dst_hbm.at[pl.ds(base + nfull * K, tail)], tidx_v)
        pltpu.sync_copy(ones_v.at[pl.ds(0, tail)],
                        shared_hist.at[tidx_v], add=True)

    plsc.subcore_barrier()
    pltpu.sync_copy(shared_hist.at[pl.ds(s * HSLICE, HSLICE)],
                    out_hbm.at[pl.ds(c * HPAD + s * HSLICE, HSLICE)])


def _degree_partials(dst):
    return pl.kernel(
        _hist_body,
        out_type=jax.ShapeDtypeStruct((NC * HPAD,), jnp.float32),
        mesh=plsc.VectorSubcoreMesh(core_axis_name="c", subcore_axis_name="s",
                                    num_cores=NC, num_subcores=NS),
        compiler_params=pltpu.CompilerParams(needs_layout_passes=False),
        scratch_types=[
            pltpu.VMEM((K,), jnp.int32),
            pltpu.VMEM((EPT - (EPT // K) * K,), jnp.int32),
            pltpu.VMEM((K,), jnp.float32),
            pltpu.VMEM((HSLICE,), jnp.float32),
            pltpu.VMEM_SHARED((HPAD,), jnp.float32),
        ],
    )(dst)


def _scatter_body(hs_hbm, src_hbm, dst_hbm, zero_hbm, acc_hbm,
                  src_v, dst_v, pend_v, lchunk_v, schunk_v, rows_v,
                  acc_shared):
    c = lax.axis_index("c")
    s = lax.axis_index("s")
    ebase = s * EPS
    lanes = lax.iota(jnp.int32, 16)
    junk = jnp.int32(JROW << 17)

    def one_pass(p, _):
        r0 = (c * PASSES + p) * R

        pltpu.sync_copy(zero_hbm, acc_shared.at[pl.ds(s * ZROWS, ZROWS)])
        plsc.subcore_barrier()

        def one_block(blk, _):
            bbase = ebase + blk * B_BLK
            pltpu.sync_copy(src_hbm.at[pl.ds(bbase, B_BLK)], src_v)
            pltpu.sync_copy(dst_hbm.at[pl.ds(bbase, B_BLK)], dst_v)

            junkv = jnp.full((16,), junk, jnp.int32)

            def scan(i, carry):
                cnt, outv = carry
                dv = dst_v[pl.ds(i * 16, 16)]
                sv = src_v[pl.ds(i * 16, 16)]
                m = (dv >= r0) & (dv < r0 + R)
                pv = ((dv - r0) << 17) | sv
                nm = plsc.all_reduce_population_count(m)[0]

                def ext(j, carry_):
                    m_, cnt_, outv_ = carry_
                    k = plsc.all_reduce_ffs(m_)[0]
                    kv = jnp.zeros((16,), jnp.int32) + k
                    valv = jnp.take_along_axis(pv, kv, axis=0)
                    slot = lax.rem(cnt_, 16)
                    outv_ = jnp.where(lanes == slot, valv, outv_)
                    pend_v[pl.ds((cnt_ // 16) * 16, 16)] = outv_
                    return (m_ & (lanes != k), cnt_ + 1, outv_)
                _, cnt, outv = lax.fori_loop(0, nm, ext, (m, cnt, outv))
                return (cnt, outv)
            cnt, outv = lax.fori_loop(
                0, NV, scan, (jnp.int32(0), junkv))

            gbase = (cnt // 16) * 16
            pend_v[pl.ds(gbase, 16)] = jnp.where(
                lanes < lax.rem(cnt, 16), outv, junkv)

            def pad(i, _):
                pend_v[pl.ds(gbase + (i + 1) * 16, 16)] = junkv
                return 0
            lax.fori_loop(0, K // 16, pad, 0)

            nsub = (cnt + (K - 1)) // K

            def drain(d, _):
                for t in range(K // 16):
                    pv = pend_v[pl.ds(d * K + t * 16, 16)]
                    schunk_v[pl.ds(t * 16, 16)] = pv & 0x1FFFF
                    lchunk_v[pl.ds(t * 16, 16)] = lax.shift_right_logical(
                        pv, 17)
                pltpu.sync_copy(hs_hbm.at[schunk_v], rows_v)
                pltpu.sync_copy(rows_v, acc_shared.at[lchunk_v], add=True)
                return 0
            lax.fori_loop(0, nsub, drain, 0)
            return 0
        lax.fori_loop(0, EPS // B_BLK, one_block, 0)

        plsc.subcore_barrier()
        pltpu.sync_copy(
            acc_shared.at[pl.ds(s * ZROWS, ZROWS)],
            acc_hbm.at[pl.ds(r0 + s * ZROWS, ZROWS)])
        return 0
    lax.fori_loop(0, PASSES, one_pass, 0)


def _scatter_rows(hs, src, dst, zero):
    return pl.kernel(
        _scatter_body,
        out_type=jax.ShapeDtypeStruct((NPAD, DIM_IN), jnp.float32),
        mesh=plsc.VectorSubcoreMesh(core_axis_name="c", subcore_axis_name="s",
                                    num_cores=NC, num_subcores=NS),
        compiler_params=pltpu.CompilerParams(needs_layout_passes=False),
        scratch_types=[
            pltpu.VMEM((B_BLK,), jnp.int32),
            pltpu.VMEM((B_BLK,), jnp.int32),
            pltpu.VMEM((CAP,), jnp.int32),
            pltpu.VMEM((K,), jnp.int32),
            pltpu.VMEM((K,), jnp.int32),
            pltpu.VMEM((K, DIM_IN), jnp.float32),
            pltpu.VMEM_SHARED((R + 8, DIM_IN), jnp.float32),
        ],
    )(hs, src, dst, zero)


ROWB = 2000


def _mlp_block(x_ref, w1_ref, b1_ref, w3_ref, b3_ref, wg_ref, degp_ref,
               hs_ref, dinv_ref):
    x = x_ref[...]
    l1 = jnp.dot(x, w1_ref[...], preferred_element_type=jnp.float32) + b1_ref[...]
    l1 = jnp.where(l1 > 0, l1, 0.01 * l1)
    l3 = jnp.dot(l1, w3_ref[...], preferred_element_type=jnp.float32) + b3_ref[...]
    l3 = jnp.where(l3 > 0, l3, 0.01 * l3)
    h = jnp.dot(l3, wg_ref[...], preferred_element_type=jnp.float32)
    deg = degp_ref[0] + degp_ref[1] + 1.0
    dinv = lax.rsqrt(deg)
    dinv_ref[...] = dinv
    hs_ref[...] = h * dinv


def _fused_mlp(x, W1, b1, W3, b3, Wg, degp):
    grid = (N // ROWB,)
    return pl.pallas_call(
        _mlp_block,
        grid=grid,
        in_specs=[
            pl.BlockSpec((ROWB, LATENT), lambda i: (i, 0)),
            pl.BlockSpec((LATENT, LATENT), lambda i: (0, 0)),
            pl.BlockSpec((1, LATENT), lambda i: (0, 0)),
            pl.BlockSpec((LATENT, DIM_HALF), lambda i: (0, 0)),
            pl.BlockSpec((1, DIM_HALF), lambda i: (0, 0)),
            pl.BlockSpec((DIM_HALF, DIM_IN), lambda i: (0, 0)),
            pl.BlockSpec((NC, ROWB, 1), lambda i: (0, i, 0)),
        ],
        out_specs=[
            pl.BlockSpec((ROWB, DIM_IN), lambda i: (i, 0)),
            pl.BlockSpec((ROWB, 1), lambda i: (i, 0)),
        ],
        out_shape=[
            jax.ShapeDtypeStruct((N, DIM_IN), jnp.float32),
            jax.ShapeDtypeStruct((N, 1), jnp.float32),
        ],
    )(x, W1, b1.reshape(1, LATENT), W3, b3.reshape(1, DIM_HALF), Wg, degp)


def _epilogue_block(acc_ref, hs_ref, dinv_ref, bg_ref, out_ref):
    v = (acc_ref[...] + hs_ref[...]) * dinv_ref[...] + bg_ref[...]
    out_ref[...] = jax.nn.sigmoid(v)


def _epilogue(acc, hs, dinv, bg):
    grid = (N // ROWB,)
    return pl.pallas_call(
        _epilogue_block,
        grid=grid,
        in_specs=[
            pl.BlockSpec((ROWB, DIM_IN), lambda i: (i, 0)),
            pl.BlockSpec((ROWB, DIM_IN), lambda i: (i, 0)),
            pl.BlockSpec((ROWB, 1), lambda i: (i, 0)),
            pl.BlockSpec((1, DIM_IN), lambda i: (0, 0)),
        ],
        out_specs=pl.BlockSpec((ROWB, DIM_IN), lambda i: (i, 0)),
        out_shape=jax.ShapeDtypeStruct((N, DIM_IN), jnp.float32),
    )(acc, hs, dinv, bg.reshape(1, DIM_IN))


def kernel(x, edge_index, batch, W1, b1, W3, b3, Wg, bg):
    del batch
    src = edge_index[0].astype(jnp.int32)
    dst = edge_index[1].astype(jnp.int32)
    degp = _degree_partials(dst).reshape(NC, HPAD)
    degp3 = degp[:, :N].reshape(NC, N, 1)
    hs, dinv = _fused_mlp(x, W1, b1, W3, b3, Wg, degp3)
    zero = jnp.zeros((ZROWS, DIM_IN), jnp.float32)
    acc = _scatter_rows(hs, src, dst, zero)
    return _epilogue(acc, hs, dinv, bg)


# speedup vs baseline: 8.2644x; 1.2610x over previous
import functools

import jax
import jax.numpy as jnp
from jax import lax
from jax.experimental import pallas as pl
from jax.experimental.pallas import tpu as pltpu
from jax.experimental.pallas import tpu_sc as plsc

N = 100000
E = 1600000
LATENT = 128
DIM_HALF = 32
DIM_IN = 128

NC = 2
NS = 16
NW = NC * NS

EPT = E // NW
HSLICE = 6272
HPAD = NS * HSLICE

R = 8448
PASSES = 6
NPAD = NC * PASSES * R
JROW = R
EPS = E // NS
B_BLK = 10000
NV = B_BLK // 16
CAP = 10144
K = 96
ZROWS = R // NS


def _zero_i32(ref, n):
    def body(i, _):
        ref[pl.ds(i * 16, 16)] = jnp.zeros((16,), jnp.int32)
        return 0
    lax.fori_loop(0, n // 16, body, 0)


def _hist_body(dst_hbm, out_hbm, idx_v, tidx_v, ones_v, zrow_v, shared_hist):
    c = lax.axis_index("c")
    s = lax.axis_index("s")
    wid = c * NS + s

    def fill_ones(i, _):
        ones_v[pl.ds(i * 16, 16)] = jnp.ones((16,), jnp.float32)
        return 0
    lax.fori_loop(0, K // 16, fill_ones, 0)

    def fill_z(i, _):
        zrow_v[pl.ds(i * 16, 16)] = jnp.zeros((16,), jnp.float32)
        return 0
    lax.fori_loop(0, HSLICE // 16, fill_z, 0)

    pltpu.sync_copy(zrow_v, shared_hist.at[pl.ds(s * HSLICE, HSLICE)])
    plsc.subcore_barrier()

    base = wid * EPT
    nfull = EPT // K

    def chunk(j, _):
        pltpu.sync_copy(dst_hbm.at[pl.ds(base + j * K, K)], idx_v)
        pltpu.sync_copy(ones_v, shared_hist.at[idx_v], add=True)
        return 0
    lax.fori_loop(0, nfull, chunk, 0)
    tail = EPT - nfull * K
    if tail:
        pltpu.sync_copy(dst_hbm.at[pl.ds(base + nfull * K, tail)], tidx_v)
        pltpu.sync_copy(ones_v.at[pl.ds(0, tail)],
                        shared_hist.at[tidx_v], add=True)

    plsc.subcore_barrier()
    pltpu.sync_copy(shared_hist.at[pl.ds(s * HSLICE, HSLICE)],
                    out_hbm.at[pl.ds(c * HPAD + s * HSLICE, HSLICE)])


def _degree_partials(dst):
    return pl.kernel(
        _hist_body,
        out_type=jax.ShapeDtypeStruct((NC * HPAD,), jnp.float32),
        mesh=plsc.VectorSubcoreMesh(core_axis_name="c", subcore_axis_name="s",
                                    num_cores=NC, num_subcores=NS),
        compiler_params=pltpu.CompilerParams(needs_layout_passes=False),
        scratch_types=[
            pltpu.VMEM((K,), jnp.int32),
            pltpu.VMEM((EPT - (EPT // K) * K,), jnp.int32),
            pltpu.VMEM((K,), jnp.float32),
            pltpu.VMEM((HSLICE,), jnp.float32),
            pltpu.VMEM_SHARED((HPAD,), jnp.float32),
        ],
    )(dst)


def _scatter_body(hs_hbm, src_hbm, dst_hbm, zero_hbm, acc_hbm,
                  src_v, dst_v, pend_v, lchunk_v, schunk_v, rows_v,
                  lchunk2_v, schunk2_v, rows2_v, gsem, gsem2,
                  acc_shared):
    c = lax.axis_index("c")
    s = lax.axis_index("s")
    ebase = s * EPS
    lanes = lax.iota(jnp.int32, 16)
    junk = jnp.int32(JROW << 17)

    def one_pass(p, _):
        r0 = (c * PASSES + p) * R

        pltpu.sync_copy(zero_hbm, acc_shared.at[pl.ds(s * ZROWS, ZROWS)])
        plsc.subcore_barrier()

        def one_block(blk, _):
            bbase = ebase + blk * B_BLK
            pltpu.sync_copy(src_hbm.at[pl.ds(bbase, B_BLK)], src_v)
            pltpu.sync_copy(dst_hbm.at[pl.ds(bbase, B_BLK)], dst_v)

            junkv = jnp.full((16,), junk, jnp.int32)

            def scan(i, carry):
                cnt, outv = carry
                dv = dst_v[pl.ds(i * 16, 16)]
                sv = src_v[pl.ds(i * 16, 16)]
                m = (dv >= r0) & (dv < r0 + R)
                pv = ((dv - r0) << 17) | sv
                nm = plsc.all_reduce_population_count(m)[0]

                def ext(j, carry_):
                    m_, cnt_, outv_ = carry_
                    k = plsc.all_reduce_ffs(m_)[0]
                    kv = jnp.zeros((16,), jnp.int32) + k
                    valv = jnp.take_along_axis(pv, kv, axis=0)
                    slot = lax.rem(cnt_, 16)
                    outv_ = jnp.where(lanes == slot, valv, outv_)
                    pend_v[pl.ds((cnt_ // 16) * 16, 16)] = outv_
                    return (m_ & (lanes != k), cnt_ + 1, outv_)
                _, cnt, outv = lax.fori_loop(0, nm, ext, (m, cnt, outv))
                return (cnt, outv)
            cnt, outv = lax.fori_loop(
                0, NV, scan, (jnp.int32(0), junkv))

            gbase = (cnt // 16) * 16
            pend_v[pl.ds(gbase, 16)] = jnp.where(
                lanes < lax.rem(cnt, 16), outv, junkv)

            def pad(i, _):
                pend_v[pl.ds(gbase + (i + 1) * 16, 16)] = junkv
                return 0
            lax.fori_loop(0, K // 16, pad, 0)

            nsub = (cnt + (K - 1)) // K

            def decode(d, sc_ref, lc_ref):
                for t in range(K // 16):
                    pv = pend_v[pl.ds(d * K + t * 16, 16)]
                    sc_ref[pl.ds(t * 16, 16)] = pv & 0x1FFFF
                    lc_ref[pl.ds(t * 16, 16)] = lax.shift_right_logical(
                        pv, 17)

            @pl.when(nsub > 0)
            def _():
                decode(0, schunk_v, lchunk_v)
                pltpu.async_copy(hs_hbm.at[schunk_v], rows_v, gsem)

            def drain(d, _):
                @pl.when(lax.rem(d, 2) == 0)
                def _():
                    pltpu.make_async_copy(
                        hs_hbm.at[schunk_v], rows_v, gsem).wait()

                    @pl.when(d + 1 < nsub)
                    def _():
                        decode(d + 1, schunk2_v, lchunk2_v)
                        pltpu.async_copy(
                            hs_hbm.at[schunk2_v], rows2_v, gsem2)
                    pltpu.sync_copy(rows_v, acc_shared.at[lchunk_v], add=True)

                @pl.when(lax.rem(d, 2) == 1)
                def _():
                    pltpu.make_async_copy(
                        hs_hbm.at[schunk2_v], rows2_v, gsem2).wait()

                    @pl.when(d + 1 < nsub)
                    def _():
                        decode(d + 1, schunk_v, lchunk_v)
                        pltpu.async_copy(hs_hbm.at[schunk_v], rows_v, gsem)
                    pltpu.sync_copy(
                        rows2_v, acc_shared.at[lchunk2_v], add=True)
                return 0
            lax.fori_loop(0, nsub, drain, 0)
            return 0
        lax.fori_loop(0, EPS // B_BLK, one_block, 0)

        plsc.subcore_barrier()
        pltpu.sync_copy(
            acc_shared.at[pl.ds(s * ZROWS, ZROWS)],
            acc_hbm.at[pl.ds(r0 + s * ZROWS, ZROWS)])
        return 0
    lax.fori_loop(0, PASSES, one_pass, 0)


def _scatter_rows(hs, src, dst, zero):
    return pl.kernel(
        _scatter_body,
        out_type=jax.ShapeDtypeStruct((NPAD, DIM_IN), jnp.float32),
        mesh=plsc.VectorSubcoreMesh(core_axis_name="c", subcore_axis_name="s",
                                    num_cores=NC, num_subcores=NS),
        compiler_params=pltpu.CompilerParams(needs_layout_passes=False),
        scratch_types=[
            pltpu.VMEM((B_BLK,), jnp.int32),
            pltpu.VMEM((B_BLK,), jnp.int32),
            pltpu.VMEM((CAP,), jnp.int32),
            pltpu.VMEM((K,), jnp.int32),
            pltpu.VMEM((K,), jnp.int32),
            pltpu.VMEM((K, DIM_IN), jnp.float32),
            pltpu.VMEM((K,), jnp.int32),
            pltpu.VMEM((K,), jnp.int32),
            pltpu.VMEM((K, DIM_IN), jnp.float32),
            pltpu.SemaphoreType.DMA,
            pltpu.SemaphoreType.DMA,
            pltpu.VMEM_SHARED((R + 8, DIM_IN), jnp.float32),
        ],
    )(hs, src, dst, zero)


ROWB = 2000


def _mlp_block(x_ref, w1_ref, b1_ref, w3_ref, b3_ref, wg_ref, degp_ref,
               hs_ref, dinv_ref):
    x = x_ref[...]
    l1 = jnp.dot(x, w1_ref[...], preferred_element_type=jnp.float32) + b1_ref[...]
    l1 = jnp.where(l1 > 0, l1, 0.01 * l1)
    l3 = jnp.dot(l1, w3_ref[...], preferred_element_type=jnp.float32) + b3_ref[...]
    l3 = jnp.where(l3 > 0, l3, 0.01 * l3)
    h = jnp.dot(l3, wg_ref[...], preferred_element_type=jnp.float32)
    deg = degp_ref[0] + degp_ref[1] + 1.0
    dinv = lax.rsqrt(deg)
    dinv_ref[...] = dinv
    hs_ref[...] = h * dinv


def _fused_mlp(x, W1, b1, W3, b3, Wg, degp):
    grid = (N // ROWB,)
    return pl.pallas_call(
        _mlp_block,
        grid=grid,
        in_specs=[
            pl.BlockSpec((ROWB, LATENT), lambda i: (i, 0)),
            pl.BlockSpec((LATENT, LATENT), lambda i: (0, 0)),
            pl.BlockSpec((1, LATENT), lambda i: (0, 0)),
            pl.BlockSpec((LATENT, DIM_HALF), lambda i: (0, 0)),
            pl.BlockSpec((1, DIM_HALF), lambda i: (0, 0)),
            pl.BlockSpec((DIM_HALF, DIM_IN), lambda i: (0, 0)),
            pl.BlockSpec((NC, ROWB, 1), lambda i: (0, i, 0)),
        ],
        out_specs=[
            pl.BlockSpec((ROWB, DIM_IN), lambda i: (i, 0)),
            pl.BlockSpec((ROWB, 1), lambda i: (i, 0)),
        ],
        out_shape=[
            jax.ShapeDtypeStruct((N, DIM_IN), jnp.float32),
            jax.ShapeDtypeStruct((N, 1), jnp.float32),
        ],
    )(x, W1, b1.reshape(1, LATENT), W3, b3.reshape(1, DIM_HALF), Wg, degp)


def _epilogue_block(acc_ref, hs_ref, dinv_ref, bg_ref, out_ref):
    v = (acc_ref[...] + hs_ref[...]) * dinv_ref[...] + bg_ref[...]
    out_ref[...] = jax.nn.sigmoid(v)


def _epilogue(acc, hs, dinv, bg):
    grid = (N // ROWB,)
    return pl.pallas_call(
        _epilogue_block,
        grid=grid,
        in_specs=[
            pl.BlockSpec((ROWB, DIM_IN), lambda i: (i, 0)),
            pl.BlockSpec((ROWB, DIM_IN), lambda i: (i, 0)),
            pl.BlockSpec((ROWB, 1), lambda i: (i, 0)),
            pl.BlockSpec((1, DIM_IN), lambda i: (0, 0)),
        ],
        out_specs=pl.BlockSpec((ROWB, DIM_IN), lambda i: (i, 0)),
        out_shape=jax.ShapeDtypeStruct((N, DIM_IN), jnp.float32),
    )(acc, hs, dinv, bg.reshape(1, DIM_IN))


def kernel(x, edge_index, batch, W1, b1, W3, b3, Wg, bg):
    del batch
    src = edge_index[0].astype(jnp.int32)
    dst = edge_index[1].astype(jnp.int32)
    degp = _degree_partials(dst).reshape(NC, HPAD)
    degp3 = degp[:, :N].reshape(NC, N, 1)
    hs, dinv = _fused_mlp(x, W1, b1, W3, b3, Wg, degp3)
    zero = jnp.zeros((ZROWS, DIM_IN), jnp.float32)
    acc = _scatter_rows(hs, src, dst, zero)
    return _epilogue(acc, hs, dinv, bg)


# speedup vs baseline: 8.2726x; 1.0010x over previous
import jax
import jax.numpy as jnp
from jax import lax
from jax.experimental import pallas as pl
from jax.experimental.pallas import tpu as pltpu
from jax.experimental.pallas import tpu_sc as plsc

N = 100000
E = 1600000
LATENT = 128
DIM_HALF = 32
DIM_IN = 128

NC = 2
NS = 16
NW = NC * NS

EPT = E // NW
HSLICE = 6272
HPAD = NS * HSLICE

R = 8448
PASSES = 6
NPAD = NC * PASSES * R
JROW = R
EPS = E // NS
B_BLK = 10000
NV = B_BLK // 16
CAP = 10144
K = 96
ZROWS = R // NS


def _zero_i32(ref, n):
    def body(i, _):
        ref[pl.ds(i * 16, 16)] = jnp.zeros((16,), jnp.int32)
        return 0
    lax.fori_loop(0, n // 16, body, 0)


def _hist_body(dst_hbm, out_hbm, idx_v, tidx_v, ones_v, zrow_v, shared_hist):
    c = lax.axis_index("c")
    s = lax.axis_index("s")
    wid = c * NS + s

    def fill_ones(i, _):
        ones_v[pl.ds(i * 16, 16)] = jnp.ones((16,), jnp.float32)
        return 0
    lax.fori_loop(0, K // 16, fill_ones, 0)

    def fill_z(i, _):
        zrow_v[pl.ds(i * 16, 16)] = jnp.zeros((16,), jnp.float32)
        return 0
    lax.fori_loop(0, HSLICE // 16, fill_z, 0)

    pltpu.sync_copy(zrow_v, shared_hist.at[pl.ds(s * HSLICE, HSLICE)])
    plsc.subcore_barrier()

    base = wid * EPT
    nfull = EPT // K

    def chunk(j, _):
        pltpu.sync_copy(dst_hbm.at[pl.ds(base + j * K, K)], idx_v)
        pltpu.sync_copy(ones_v, shared_hist.at[idx_v], add=True)
        return 0
    lax.fori_loop(0, nfull, chunk, 0)
    tail = EPT - nfull * K
    if tail:
        pltpu.sync_copy(dst_hbm.at[pl.ds(base + nfull * K, tail)], tidx_v)
        pltpu.sync_copy(ones_v.at[pl.ds(0, tail)],
                        shared_hist.at[tidx_v], add=True)

    plsc.subcore_barrier()
    pltpu.sync_copy(shared_hist.at[pl.ds(s * HSLICE, HSLICE)],
                    out_hbm.at[pl.ds(c * HPAD + s * HSLICE, HSLICE)])


def _degree_partials(dst):
    return pl.kernel(
        _hist_body,
        out_type=jax.ShapeDtypeStruct((NC * HPAD,), jnp.float32),
        mesh=plsc.VectorSubcoreMesh(core_axis_name="c", subcore_axis_name="s",
                                    num_cores=NC, num_subcores=NS),
        compiler_params=pltpu.CompilerParams(needs_layout_passes=False),
        scratch_types=[
            pltpu.VMEM((K,), jnp.int32),
            pltpu.VMEM((EPT - (EPT // K) * K,), jnp.int32),
            pltpu.VMEM((K,), jnp.float32),
            pltpu.VMEM((HSLICE,), jnp.float32),
            pltpu.VMEM_SHARED((HPAD,), jnp.float32),
        ],
    )(dst)


def _scatter_body(hs_hbm, src_hbm, dst_hbm, zero_hbm, acc_hbm,
                  src_v, dst_v, pend_v, lchunk_v, schunk_v, rows_v,
                  lchunk2_v, schunk2_v, rows2_v, gsem, gsem2,
                  acc_shared):
    c = lax.axis_index("c")
    s = lax.axis_index("s")
    ebase = s * EPS
    lanes = lax.iota(jnp.int32, 16)
    junk = jnp.int32(JROW << 17)

    def one_pass(p, _):
        r0 = (c * PASSES + p) * R

        pltpu.sync_copy(zero_hbm, acc_shared.at[pl.ds(s * ZROWS, ZROWS)])
        plsc.subcore_barrier()

        def one_block(blk, _):
            bbase = ebase + blk * B_BLK
            pltpu.sync_copy(src_hbm.at[pl.ds(bbase, B_BLK)], src_v)
            pltpu.sync_copy(dst_hbm.at[pl.ds(bbase, B_BLK)], dst_v)

            junkv = jnp.full((16,), junk, jnp.int32)

            def scan(i, carry):
                cnt, outv = carry
                dv = dst_v[pl.ds(i * 16, 16)]
                sv = src_v[pl.ds(i * 16, 16)]
                m = (dv >= r0) & (dv < r0 + R)
                pv = ((dv - r0) << 17) | sv
                nm = plsc.all_reduce_population_count(m)[0]

                def ext(j, carry_):
                    m_, cnt_, outv_ = carry_
                    k = plsc.all_reduce_ffs(m_)[0]
                    kv = jnp.zeros((16,), jnp.int32) + k
                    valv = jnp.take_along_axis(pv, kv, axis=0)
                    slot = lax.rem(cnt_, 16)
                    outv_ = jnp.where(lanes == slot, valv, outv_)
                    pend_v[pl.ds((cnt_ // 16) * 16, 16)] = outv_
                    return (m_ & (lanes != k), cnt_ + 1, outv_)
                _, cnt, outv = lax.fori_loop(0, nm, ext, (m, cnt, outv))
                return (cnt, outv)
            cnt, outv = lax.fori_loop(
                0, NV, scan, (jnp.int32(0), junkv))

            gbase = (cnt // 16) * 16
            pend_v[pl.ds(gbase, 16)] = jnp.where(
                lanes < lax.rem(cnt, 16), outv, junkv)

            def pad(i, _):
                pend_v[pl.ds(gbase + (i + 1) * 16, 16)] = junkv
                return 0
            lax.fori_loop(0, K // 16, pad, 0)

            nsub = (cnt + (K - 1)) // K

            def decode(d, sc_ref, lc_ref):
                for t in range(K // 16):
                    pv = pend_v[pl.ds(d * K + t * 16, 16)]
                    sc_ref[pl.ds(t * 16, 16)] = pv & 0x1FFFF
                    lc_ref[pl.ds(t * 16, 16)] = lax.shift_right_logical(
                        pv, 17)

            @pl.when(nsub > 0)
            def _():
                decode(0, schunk_v, lchunk_v)
                pltpu.async_copy(hs_hbm.at[schunk_v], rows_v, gsem)

            def drain(d, _):
                @pl.when(lax.rem(d, 2) == 0)
                def _():
                    pltpu.make_async_copy(
                        hs_hbm.at[schunk_v], rows_v, gsem).wait()

                    @pl.when(d + 1 < nsub)
                    def _():
                        decode(d + 1, schunk2_v, lchunk2_v)
                        pltpu.async_copy(
                            hs_hbm.at[schunk2_v], rows2_v, gsem2)
                    pltpu.sync_copy(rows_v, acc_shared.at[lchunk_v], add=True)

                @pl.when(lax.rem(d, 2) == 1)
                def _():
                    pltpu.make_async_copy(
                        hs_hbm.at[schunk2_v], rows2_v, gsem2).wait()

                    @pl.when(d + 1 < nsub)
                    def _():
                        decode(d + 1, schunk_v, lchunk_v)
                        pltpu.async_copy(hs_hbm.at[schunk_v], rows_v, gsem)
                    pltpu.sync_copy(
                        rows2_v, acc_shared.at[lchunk2_v], add=True)
                return 0
            lax.fori_loop(0, nsub, drain, 0)
            return 0
        lax.fori_loop(0, EPS // B_BLK, one_block, 0)

        plsc.subcore_barrier()
        pltpu.sync_copy(
            acc_shared.at[pl.ds(s * ZROWS, ZROWS)],
            acc_hbm.at[pl.ds(r0 + s * ZROWS, ZROWS)])
        return 0
    lax.fori_loop(0, PASSES, one_pass, 0)


def _scatter_rows(hs, src, dst, zero):
    return pl.kernel(
        _scatter_body,
        out_type=jax.ShapeDtypeStruct((NPAD, DIM_IN), jnp.float32),
        mesh=plsc.VectorSubcoreMesh(core_axis_name="c", subcore_axis_name="s",
                                    num_cores=NC, num_subcores=NS),
        compiler_params=pltpu.CompilerParams(needs_layout_passes=False),
        scratch_types=[
            pltpu.VMEM((B_BLK,), jnp.int32),
            pltpu.VMEM((B_BLK,), jnp.int32),
            pltpu.VMEM((CAP,), jnp.int32),
            pltpu.VMEM((K,), jnp.int32),
            pltpu.VMEM((K,), jnp.int32),
            pltpu.VMEM((K, DIM_IN), jnp.float32),
            pltpu.VMEM((K,), jnp.int32),
            pltpu.VMEM((K,), jnp.int32),
            pltpu.VMEM((K, DIM_IN), jnp.float32),
            pltpu.SemaphoreType.DMA,
            pltpu.SemaphoreType.DMA,
            pltpu.VMEM_SHARED((R + 8, DIM_IN), jnp.float32),
        ],
    )(hs, src, dst, zero)


ROWB = 2000


def _mlp_block(x_ref, w1_ref, b1_ref, w3_ref, b3_ref, wg_ref, degp_ref,
               hs_ref, dinv_ref):
    x = x_ref[...]
    l1 = jnp.dot(x, w1_ref[...], preferred_element_type=jnp.float32) + b1_ref[...]
    l1 = jnp.where(l1 > 0, l1, 0.01 * l1)
    l3 = jnp.dot(l1, w3_ref[...], preferred_element_type=jnp.float32) + b3_ref[...]
    l3 = jnp.where(l3 > 0, l3, 0.01 * l3)
    h = jnp.dot(l3, wg_ref[...], preferred_element_type=jnp.float32)
    deg = degp_ref[0] + degp_ref[1] + 1.0
    dinv = lax.rsqrt(deg)
    dinv_ref[...] = dinv
    hs_ref[...] = h * dinv


def _fused_mlp(x, W1, b1, W3, b3, Wg, degp):
    grid = (N // ROWB,)
    return pl.pallas_call(
        _mlp_block,
        grid=grid,
        in_specs=[
            pl.BlockSpec((ROWB, LATENT), lambda i: (i, 0)),
            pl.BlockSpec((LATENT, LATENT), lambda i: (0, 0)),
            pl.BlockSpec((1, LATENT), lambda i: (0, 0)),
            pl.BlockSpec((LATENT, DIM_HALF), lambda i: (0, 0)),
            pl.BlockSpec((1, DIM_HALF), lambda i: (0, 0)),
            pl.BlockSpec((DIM_HALF, DIM_IN), lambda i: (0, 0)),
            pl.BlockSpec((NC, ROWB, 1), lambda i: (0, i, 0)),
        ],
        out_specs=[
            pl.BlockSpec((ROWB, DIM_IN), lambda i: (i, 0)),
            pl.BlockSpec((ROWB, 1), lambda i: (i, 0)),
        ],
        out_shape=[
            jax.ShapeDtypeStruct((N, DIM_IN), jnp.float32),
            jax.ShapeDtypeStruct((N, 1), jnp.float32),
        ],
    )(x, W1, b1.reshape(1, LATENT), W3, b3.reshape(1, DIM_HALF), Wg, degp)


def _epilogue_block(acc_ref, hs_ref, dinv_ref, bg_ref, out_ref):
    v = (acc_ref[...] + hs_ref[...]) * dinv_ref[...] + bg_ref[...]
    out_ref[...] = jax.nn.sigmoid(v)


def _epilogue(acc, hs, dinv, bg):
    grid = (N // ROWB,)
    return pl.pallas_call(
        _epilogue_block,
        grid=grid,
        in_specs=[
            pl.BlockSpec((ROWB, DIM_IN), lambda i: (i, 0)),
            pl.BlockSpec((ROWB, DIM_IN), lambda i: (i, 0)),
            pl.BlockSpec((ROWB, 1), lambda i: (i, 0)),
            pl.BlockSpec((1, DIM_IN), lambda i: (0, 0)),
        ],
        out_specs=pl.BlockSpec((ROWB, DIM_IN), lambda i: (i, 0)),
        out_shape=jax.ShapeDtypeStruct((N, DIM_IN), jnp.float32),
    )(acc, hs, dinv, bg.reshape(1, DIM_IN))


def kernel(x, edge_index, batch, W1, b1, W3, b3, Wg, bg):
    del batch
    src = edge_index[0].astype(jnp.int32)
    dst = edge_index[1].astype(jnp.int32)
    degp = _degree_partials(dst).reshape(NC, HPAD)
    degp3 = degp[:, :N].reshape(NC, N, 1)
    hs, dinv = _fused_mlp(x, W1, b1, W3, b3, Wg, degp3)
    zero = jnp.zeros((ZROWS, DIM_IN), jnp.float32)
    acc = _scatter_rows(hs, src, dst, zero)
    return _epilogue(acc, hs, dinv, bg)


# speedup vs baseline: 8.4105x; 1.0167x over previous
import jax
import jax.numpy as jnp
from jax import lax
from jax.experimental import pallas as pl
from jax.experimental.pallas import tpu as pltpu
from jax.experimental.pallas import tpu_sc as plsc

N = 100000
E = 1600000
LATENT = 128
DIM_HALF = 32
DIM_IN = 128

NC = 2
NS = 16
NW = NC * NS

EPT = E // NW
HSLICE = 6272
HPAD = NS * HSLICE

R = 8448
PASSES = 6
NPAD = NC * PASSES * R
JROW = R
EPS = E // NS
B_BLK = 10000
NV = B_BLK // 16
CAP = 10144
K = 96
ZROWS = R // NS


def _zero_i32(ref, n):
    def body(i, _):
        ref[pl.ds(i * 16, 16)] = jnp.zeros((16,), jnp.int32)
        return 0
    lax.fori_loop(0, n // 16, body, 0)


def _hist_body(dst_hbm, out_hbm, idx_v, tidx_v, ones_v, zrow_v, shared_hist):
    c = lax.axis_index("c")
    s = lax.axis_index("s")
    wid = c * NS + s

    def fill_ones(i, _):
        ones_v[pl.ds(i * 16, 16)] = jnp.ones((16,), jnp.float32)
        return 0
    lax.fori_loop(0, K // 16, fill_ones, 0)

    def fill_z(i, _):
        zrow_v[pl.ds(i * 16, 16)] = jnp.zeros((16,), jnp.float32)
        return 0
    lax.fori_loop(0, HSLICE // 16, fill_z, 0)

    pltpu.sync_copy(zrow_v, shared_hist.at[pl.ds(s * HSLICE, HSLICE)])
    plsc.subcore_barrier()

    base = wid * EPT
    nfull = EPT // K

    def chunk(j, _):
        pltpu.sync_copy(dst_hbm.at[pl.ds(base + j * K, K)], idx_v)
        pltpu.sync_copy(ones_v, shared_hist.at[idx_v], add=True)
        return 0
    lax.fori_loop(0, nfull, chunk, 0)
    tail = EPT - nfull * K
    if tail:
        pltpu.sync_copy(dst_hbm.at[pl.ds(base + nfull * K, tail)], tidx_v)
        pltpu.sync_copy(ones_v.at[pl.ds(0, tail)],
                        shared_hist.at[tidx_v], add=True)

    plsc.subcore_barrier()
    pltpu.sync_copy(shared_hist.at[pl.ds(s * HSLICE, HSLICE)],
                    out_hbm.at[pl.ds(c * HPAD + s * HSLICE, HSLICE)])


def _degree_partials(dst):
    return pl.kernel(
        _hist_body,
        out_type=jax.ShapeDtypeStruct((NC * HPAD,), jnp.float32),
        mesh=plsc.VectorSubcoreMesh(core_axis_name="c", subcore_axis_name="s",
                                    num_cores=NC, num_subcores=NS),
        compiler_params=pltpu.CompilerParams(needs_layout_passes=False),
        scratch_types=[
            pltpu.VMEM((K,), jnp.int32),
            pltpu.VMEM((EPT - (EPT // K) * K,), jnp.int32),
            pltpu.VMEM((K,), jnp.float32),
            pltpu.VMEM((HSLICE,), jnp.float32),
            pltpu.VMEM_SHARED((HPAD,), jnp.float32),
        ],
    )(dst)


def _scatter_body(hs_hbm, src_hbm, dst_hbm, zero_hbm, acc_hbm,
                  src_v, dst_v, pend_v, lchunk_v, schunk_v, rows_v,
                  lchunk2_v, schunk2_v, rows2_v, gsem, gsem2,
                  acc_shared):
    c = lax.axis_index("c")
    s = lax.axis_index("s")
    ebase = s * EPS
    lanes = lax.iota(jnp.int32, 16)
    junk = jnp.int32(JROW << 17)

    def one_pass(p, _):
        r0 = (c * PASSES + p) * R

        pltpu.sync_copy(zero_hbm, acc_shared.at[pl.ds(s * ZROWS, ZROWS)])
        plsc.subcore_barrier()

        def one_block(blk, _):
            bbase = ebase + blk * B_BLK
            pltpu.sync_copy(src_hbm.at[pl.ds(bbase, B_BLK)], src_v)
            pltpu.sync_copy(dst_hbm.at[pl.ds(bbase, B_BLK)], dst_v)

            junkv = jnp.full((16,), junk, jnp.int32)

            def extract(m, pv, nm, cnt, outv):
                def ext(j, carry_):
                    m_, cnt_, outv_ = carry_
                    k = plsc.all_reduce_ffs(m_)[0]
                    kv = jnp.zeros((16,), jnp.int32) + k
                    valv = jnp.take_along_axis(pv, kv, axis=0)
                    slot = lax.rem(cnt_, 16)
                    outv_ = jnp.where(lanes == slot, valv, outv_)
                    pend_v[pl.ds((cnt_ // 16) * 16, 16)] = outv_
                    return (m_ & (lanes != k), cnt_ + 1, outv_)
                _, cnt, outv = lax.fori_loop(0, nm, ext, (m, cnt, outv))
                return cnt, outv

            def scan2(i, carry):
                cnt, outv = carry
                b0 = i * 32
                dv0 = dst_v[pl.ds(b0, 16)]
                sv0 = src_v[pl.ds(b0, 16)]
                dv1 = dst_v[pl.ds(b0 + 16, 16)]
                sv1 = src_v[pl.ds(b0 + 16, 16)]
                m0 = (dv0 >= r0) & (dv0 < r0 + R)
                m1 = (dv1 >= r0) & (dv1 < r0 + R)
                pv0 = ((dv0 - r0) << 17) | sv0
                pv1 = ((dv1 - r0) << 17) | sv1
                nm0 = plsc.all_reduce_population_count(m0)[0]
                nm1 = plsc.all_reduce_population_count(m1)[0]
                cnt, outv = extract(m0, pv0, nm0, cnt, outv)
                cnt, outv = extract(m1, pv1, nm1, cnt, outv)
                return (cnt, outv)
            cnt, outv = lax.fori_loop(
                0, NV // 2, scan2, (jnp.int32(0), junkv))
            dv = dst_v[pl.ds((NV - 1) * 16, 16)]
            sv = src_v[pl.ds((NV - 1) * 16, 16)]
            m = (dv >= r0) & (dv < r0 + R)
            pv = ((dv - r0) << 17) | sv
            nm = plsc.all_reduce_population_count(m)[0]
            cnt, outv = extract(m, pv, nm, cnt, outv)

            gbase = (cnt // 16) * 16
            pend_v[pl.ds(gbase, 16)] = jnp.where(
                lanes < lax.rem(cnt, 16), outv, junkv)

            def pad(i, _):
                pend_v[pl.ds(gbase + (i + 1) * 16, 16)] = junkv
                return 0
            lax.fori_loop(0, K // 16, pad, 0)

            nsub = (cnt + (K - 1)) // K

            def decode(d, sc_ref, lc_ref):
                for t in range(K // 16):
                    pv = pend_v[pl.ds(d * K + t * 16, 16)]
                    sc_ref[pl.ds(t * 16, 16)] = pv & 0x1FFFF
                    lc_ref[pl.ds(t * 16, 16)] = lax.shift_right_logical(
                        pv, 17)

            @pl.when(nsub > 0)
            def _():
                decode(0, schunk_v, lchunk_v)
                pltpu.async_copy(hs_hbm.at[schunk_v], rows_v, gsem)

            def drain(d, _):
                @pl.when(lax.rem(d, 2) == 0)
                def _():
                    pltpu.make_async_copy(
                        hs_hbm.at[schunk_v], rows_v, gsem).wait()

                    @pl.when(d + 1 < nsub)
                    def _():
                        decode(d + 1, schunk2_v, lchunk2_v)
                        pltpu.async_copy(
                            hs_hbm.at[schunk2_v], rows2_v, gsem2)
                    pltpu.sync_copy(rows_v, acc_shared.at[lchunk_v], add=True)

                @pl.when(lax.rem(d, 2) == 1)
                def _():
                    pltpu.make_async_copy(
                        hs_hbm.at[schunk2_v], rows2_v, gsem2).wait()

                    @pl.when(d + 1 < nsub)
                    def _():
                        decode(d + 1, schunk_v, lchunk_v)
                        pltpu.async_copy(hs_hbm.at[schunk_v], rows_v, gsem)
                    pltpu.sync_copy(
                        rows2_v, acc_shared.at[lchunk2_v], add=True)
                return 0
            lax.fori_loop(0, nsub, drain, 0)
            return 0
        lax.fori_loop(0, EPS // B_BLK, one_block, 0)

        plsc.subcore_barrier()
        pltpu.sync_copy(
            acc_shared.at[pl.ds(s * ZROWS, ZROWS)],
            acc_hbm.at[pl.ds(r0 + s * ZROWS, ZROWS)])
        return 0
    lax.fori_loop(0, PASSES, one_pass, 0)


def _scatter_rows(hs, src, dst, zero):
    return pl.kernel(
        _scatter_body,
        out_type=jax.ShapeDtypeStruct((NPAD, DIM_IN), jnp.float32),
        mesh=plsc.VectorSubcoreMesh(core_axis_name="c", subcore_axis_name="s",
                                    num_cores=NC, num_subcores=NS),
        compiler_params=pltpu.CompilerParams(needs_layout_passes=False),
        scratch_types=[
            pltpu.VMEM((B_BLK,), jnp.int32),
            pltpu.VMEM((B_BLK,), jnp.int32),
            pltpu.VMEM((CAP,), jnp.int32),
            pltpu.VMEM((K,), jnp.int32),
            pltpu.VMEM((K,), jnp.int32),
            pltpu.VMEM((K, DIM_IN), jnp.float32),
            pltpu.VMEM((K,), jnp.int32),
            pltpu.VMEM((K,), jnp.int32),
            pltpu.VMEM((K, DIM_IN), jnp.float32),
            pltpu.SemaphoreType.DMA,
            pltpu.SemaphoreType.DMA,
            pltpu.VMEM_SHARED((R + 8, DIM_IN), jnp.float32),
        ],
    )(hs, src, dst, zero)


ROWB = 2000


def _mlp_block(x_ref, w1_ref, b1_ref, w3_ref, b3_ref, wg_ref, h_ref):
    x = x_ref[...]
    l1 = jnp.dot(x, w1_ref[...], preferred_element_type=jnp.float32) + b1_ref[...]
    l1 = jnp.where(l1 > 0, l1, 0.01 * l1)
    l3 = jnp.dot(l1, w3_ref[...], preferred_element_type=jnp.float32) + b3_ref[...]
    l3 = jnp.where(l3 > 0, l3, 0.01 * l3)
    h_ref[...] = jnp.dot(l3, wg_ref[...], preferred_element_type=jnp.float32)


def _fused_mlp(x, W1, b1, W3, b3, Wg):
    grid = (N // ROWB,)
    return pl.pallas_call(
        _mlp_block,
        grid=grid,
        in_specs=[
            pl.BlockSpec((ROWB, LATENT), lambda i: (i, 0)),
            pl.BlockSpec((LATENT, LATENT), lambda i: (0, 0)),
            pl.BlockSpec((1, LATENT), lambda i: (0, 0)),
            pl.BlockSpec((LATENT, DIM_HALF), lambda i: (0, 0)),
            pl.BlockSpec((1, DIM_HALF), lambda i: (0, 0)),
            pl.BlockSpec((DIM_HALF, DIM_IN), lambda i: (0, 0)),
        ],
        out_specs=pl.BlockSpec((ROWB, DIM_IN), lambda i: (i, 0)),
        out_shape=jax.ShapeDtypeStruct((N, DIM_IN), jnp.float32),
    )(x, W1, b1.reshape(1, LATENT), W3, b3.reshape(1, DIM_HALF), Wg)


def _scale_block(h_ref, degp_ref, hs_ref, dinv_ref):
    deg = degp_ref[0] + degp_ref[1] + 1.0
    dinv = lax.rsqrt(deg)
    dinv_ref[...] = dinv
    hs_ref[...] = h_ref[...] * dinv


def _scale_rows(h, degp):
    grid = (N // ROWB,)
    return pl.pallas_call(
        _scale_block,
        grid=grid,
        in_specs=[
            pl.BlockSpec((ROWB, DIM_IN), lambda i: (i, 0)),
            pl.BlockSpec((NC, ROWB, 1), lambda i: (0, i, 0)),
        ],
        out_specs=[
            pl.BlockSpec((ROWB, DIM_IN), lambda i: (i, 0)),
            pl.BlockSpec((ROWB, 1), lambda i: (i, 0)),
        ],
        out_shape=[
            jax.ShapeDtypeStruct((N, DIM_IN), jnp.float32),
            jax.ShapeDtypeStruct((N, 1), jnp.float32),
        ],
    )(h, degp)


def _epilogue_block(acc_ref, hs_ref, dinv_ref, bg_ref, out_ref):
    v = (acc_ref[...] + hs_ref[...]) * dinv_ref[...] + bg_ref[...]
    out_ref[...] = jax.nn.sigmoid(v)


def _epilogue(acc, hs, dinv, bg):
    grid = (N // ROWB,)
    return pl.pallas_call(
        _epilogue_block,
        grid=grid,
        in_specs=[
            pl.BlockSpec((ROWB, DIM_IN), lambda i: (i, 0)),
            pl.BlockSpec((ROWB, DIM_IN), lambda i: (i, 0)),
            pl.BlockSpec((ROWB, 1), lambda i: (i, 0)),
            pl.BlockSpec((1, DIM_IN), lambda i: (0, 0)),
        ],
        out_specs=pl.BlockSpec((ROWB, DIM_IN), lambda i: (i, 0)),
        out_shape=jax.ShapeDtypeStruct((N, DIM_IN), jnp.float32),
    )(acc, hs, dinv, bg.reshape(1, DIM_IN))


def kernel(x, edge_index, batch, W1, b1, W3, b3, Wg, bg):
    del batch
    src = edge_index[0].astype(jnp.int32)
    dst = edge_index[1].astype(jnp.int32)
    degp = _degree_partials(dst).reshape(NC, HPAD)
    degp3 = degp[:, :N].reshape(NC, N, 1)
    h = _fused_mlp(x, W1, b1, W3, b3, Wg)
    hs, dinv = _scale_rows(h, degp3)
    zero = jnp.zeros((ZROWS, DIM_IN), jnp.float32)
    acc = _scatter_rows(hs, src, dst, zero)
    return _epilogue(acc, hs, dinv, bg)


# speedup vs baseline: 8.5825x; 1.0205x over previous
import jax
import jax.numpy as jnp
from jax import lax
from jax.experimental import pallas as pl
from jax.experimental.pallas import tpu as pltpu
from jax.experimental.pallas import tpu_sc as plsc

N = 100000
E = 1600000
LATENT = 128
DIM_HALF = 32
DIM_IN = 128

NC = 2
NS = 16
NW = NC * NS

EPT = E // NW
HK = 128
HSLICE = 6272
HPAD = NS * HSLICE

R = 8448
PASSES = 6
NPAD = NC * PASSES * R
JROW = R
EPS = E // NS
B_BLK = 10000
NV = B_BLK // 16
CAP = 10144
K = 96
ZROWS = R // NS


def _zero_i32(ref, n):
    def body(i, _):
        ref[pl.ds(i * 16, 16)] = jnp.zeros((16,), jnp.int32)
        return 0
    lax.fori_loop(0, n // 16, body, 0)


def _hist_body(dst_hbm, out_hbm, idx_v, idx2_v, tidx_v, ones_v, zrow_v,
               hsem, hsem2, shared_hist):
    c = lax.axis_index("c")
    s = lax.axis_index("s")
    wid = c * NS + s

    def fill_ones(i, _):
        ones_v[pl.ds(i * 16, 16)] = jnp.ones((16,), jnp.float32)
        return 0
    lax.fori_loop(0, HK // 16, fill_ones, 0)

    def fill_z(i, _):
        zrow_v[pl.ds(i * 16, 16)] = jnp.zeros((16,), jnp.float32)
        return 0
    lax.fori_loop(0, HSLICE // 16, fill_z, 0)

    pltpu.sync_copy(zrow_v, shared_hist.at[pl.ds(s * HSLICE, HSLICE)])
    plsc.subcore_barrier()

    base = wid * EPT
    nfull = EPT // HK

    pltpu.async_copy(dst_hbm.at[pl.ds(base, HK)], idx_v, hsem)

    def chunk(j, _):
        @pl.when(lax.rem(j, 2) == 0)
        def _():
            pltpu.make_async_copy(
                dst_hbm.at[pl.ds(base + j * HK, HK)], idx_v, hsem).wait()

            @pl.when(j + 1 < nfull)
            def _():
                pltpu.async_copy(
                    dst_hbm.at[pl.ds(base + (j + 1) * HK, HK)], idx2_v,
                    hsem2)
            pltpu.sync_copy(ones_v, shared_hist.at[idx_v], add=True)

        @pl.when(lax.rem(j, 2) == 1)
        def _():
            pltpu.make_async_copy(
                dst_hbm.at[pl.ds(base + j * HK, HK)], idx2_v, hsem2).wait()

            @pl.when(j + 1 < nfull)
            def _():
                pltpu.async_copy(
                    dst_hbm.at[pl.ds(base + (j + 1) * HK, HK)], idx_v, hsem)
            pltpu.sync_copy(ones_v, shared_hist.at[idx2_v], add=True)
        return 0
    lax.fori_loop(0, nfull, chunk, 0)
    tail = EPT - nfull * HK
    if tail:
        pltpu.sync_copy(dst_hbm.at[pl.ds(base + nfull * HK, tail)], tidx_v)
        pltpu.sync_copy(ones_v.at[pl.ds(0, tail)],
                        shared_hist.at[tidx_v], add=True)

    plsc.subcore_barrier()
    pltpu.sync_copy(shared_hist.at[pl.ds(s * HSLICE, HSLICE)],
                    out_hbm.at[pl.ds(c * HPAD + s * HSLICE, HSLICE)])


def _degree_partials(dst):
    return pl.kernel(
        _hist_body,
        out_type=jax.ShapeDtypeStruct((NC * HPAD,), jnp.float32),
        mesh=plsc.VectorSubcoreMesh(core_axis_name="c", subcore_axis_name="s",
                                    num_cores=NC, num_subcores=NS),
        compiler_params=pltpu.CompilerParams(needs_layout_passes=False),
        scratch_types=[
            pltpu.VMEM((HK,), jnp.int32),
            pltpu.VMEM((HK,), jnp.int32),
            pltpu.VMEM((EPT - (EPT // HK) * HK,), jnp.int32),
            pltpu.VMEM((HK,), jnp.float32),
            pltpu.VMEM((HSLICE,), jnp.float32),
            pltpu.SemaphoreType.DMA,
            pltpu.SemaphoreType.DMA,
            pltpu.VMEM_SHARED((HPAD,), jnp.float32),
        ],
    )(dst)


def _scatter_body(hs_hbm, src_hbm, dst_hbm, zero_hbm, acc_hbm,
                  src_v, dst_v, pend_v, lchunk_v, schunk_v, rows_v,
                  lchunk2_v, schunk2_v, rows2_v, gsem, gsem2,
                  acc_shared):
    c = lax.axis_index("c")
    s = lax.axis_index("s")
    ebase = s * EPS
    lanes = lax.iota(jnp.int32, 16)
    junk = jnp.int32(JROW << 17)

    def one_pass(p, _):
        r0 = (c * PASSES + p) * R

        pltpu.sync_copy(zero_hbm, acc_shared.at[pl.ds(s * ZROWS, ZROWS)])
        plsc.subcore_barrier()

        def one_block(blk, _):
            bbase = ebase + blk * B_BLK
            pltpu.sync_copy(src_hbm.at[pl.ds(bbase, B_BLK)], src_v)
            pltpu.sync_copy(dst_hbm.at[pl.ds(bbase, B_BLK)], dst_v)

            junkv = jnp.full((16,), junk, jnp.int32)

            def extract(m, pv, nm, cnt, outv):
                def ext(j, carry_):
                    m_, cnt_, outv_ = carry_
                    k = plsc.all_reduce_ffs(m_)[0]
                    kv = jnp.zeros((16,), jnp.int32) + k
                    valv = jnp.take_along_axis(pv, kv, axis=0)
                    slot = lax.rem(cnt_, 16)
                    outv_ = jnp.where(lanes == slot, valv, outv_)
                    pend_v[pl.ds((cnt_ // 16) * 16, 16)] = outv_
                    return (m_ & (lanes != k), cnt_ + 1, outv_)
                _, cnt, outv = lax.fori_loop(0, nm, ext, (m, cnt, outv))
                return cnt, outv

            def scan2(i, carry):
                cnt, outv = carry
                b0 = i * 32
                dv0 = dst_v[pl.ds(b0, 16)]
                sv0 = src_v[pl.ds(b0, 16)]
                dv1 = dst_v[pl.ds(b0 + 16, 16)]
                sv1 = src_v[pl.ds(b0 + 16, 16)]
                m0 = (dv0 >= r0) & (dv0 < r0 + R)
                m1 = (dv1 >= r0) & (dv1 < r0 + R)
                pv0 = ((dv0 - r0) << 17) | sv0
                pv1 = ((dv1 - r0) << 17) | sv1
                nm0 = plsc.all_reduce_population_count(m0)[0]
                nm1 = plsc.all_reduce_population_count(m1)[0]
                cnt, outv = extract(m0, pv0, nm0, cnt, outv)
                cnt, outv = extract(m1, pv1, nm1, cnt, outv)
                return (cnt, outv)
            cnt, outv = lax.fori_loop(
                0, NV // 2, scan2, (jnp.int32(0), junkv))
            dv = dst_v[pl.ds((NV - 1) * 16, 16)]
            sv = src_v[pl.ds((NV - 1) * 16, 16)]
            m = (dv >= r0) & (dv < r0 + R)
            pv = ((dv - r0) << 17) | sv
            nm = plsc.all_reduce_population_count(m)[0]
            cnt, outv = extract(m, pv, nm, cnt, outv)

            gbase = (cnt // 16) * 16
            pend_v[pl.ds(gbase, 16)] = jnp.where(
                lanes < lax.rem(cnt, 16), outv, junkv)

            def pad(i, _):
                pend_v[pl.ds(gbase + (i + 1) * 16, 16)] = junkv
                return 0
            lax.fori_loop(0, K // 16, pad, 0)

            nsub = (cnt + (K - 1)) // K

            def decode(d, sc_ref, lc_ref):
                for t in range(K // 16):
                    pv = pend_v[pl.ds(d * K + t * 16, 16)]
                    sc_ref[pl.ds(t * 16, 16)] = pv & 0x1FFFF
                    lc_ref[pl.ds(t * 16, 16)] = lax.shift_right_logical(
                        pv, 17)

            @pl.when(nsub > 0)
            def _():
                decode(0, schunk_v, lchunk_v)
                pltpu.async_copy(hs_hbm.at[schunk_v], rows_v, gsem)

            def drain(d, _):
                @pl.when(lax.rem(d, 2) == 0)
                def _():
                    pltpu.make_async_copy(
                        hs_hbm.at[schunk_v], rows_v, gsem).wait()

                    @pl.when(d + 1 < nsub)
                    def _():
                        decode(d + 1, schunk2_v, lchunk2_v)
                        pltpu.async_copy(
                            hs_hbm.at[schunk2_v], rows2_v, gsem2)
                    pltpu.sync_copy(rows_v, acc_shared.at[lchunk_v], add=True)

                @pl.when(lax.rem(d, 2) == 1)
                def _():
                    pltpu.make_async_copy(
                        hs_hbm.at[schunk2_v], rows2_v, gsem2).wait()

                    @pl.when(d + 1 < nsub)
                    def _():
                        decode(d + 1, schunk_v, lchunk_v)
                        pltpu.async_copy(hs_hbm.at[schunk_v], rows_v, gsem)
                    pltpu.sync_copy(
                        rows2_v, acc_shared.at[lchunk2_v], add=True)
                return 0
            lax.fori_loop(0, nsub, drain, 0)
            return 0
        lax.fori_loop(0, EPS // B_BLK, one_block, 0)

        plsc.subcore_barrier()
        pltpu.sync_copy(
            acc_shared.at[pl.ds(s * ZROWS, ZROWS)],
            acc_hbm.at[pl.ds(r0 + s * ZROWS, ZROWS)])
        return 0
    lax.fori_loop(0, PASSES, one_pass, 0)


def _scatter_rows(hs, src, dst, zero):
    return pl.kernel(
        _scatter_body,
        out_type=jax.ShapeDtypeStruct((NPAD, DIM_IN), jnp.float32),
        mesh=plsc.VectorSubcoreMesh(core_axis_name="c", subcore_axis_name="s",
                                    num_cores=NC, num_subcores=NS),
        compiler_params=pltpu.CompilerParams(needs_layout_passes=False),
        scratch_types=[
            pltpu.VMEM((B_BLK,), jnp.int32),
            pltpu.VMEM((B_BLK,), jnp.int32),
            pltpu.VMEM((CAP,), jnp.int32),
            pltpu.VMEM((K,), jnp.int32),
            pltpu.VMEM((K,), jnp.int32),
            pltpu.VMEM((K, DIM_IN), jnp.float32),
            pltpu.VMEM((K,), jnp.int32),
            pltpu.VMEM((K,), jnp.int32),
            pltpu.VMEM((K, DIM_IN), jnp.float32),
            pltpu.SemaphoreType.DMA,
            pltpu.SemaphoreType.DMA,
            pltpu.VMEM_SHARED((R + 8, DIM_IN), jnp.float32),
        ],
    )(hs, src, dst, zero)


ROWB = 2000


def _mlp_block(x_ref, w1_ref, b1_ref, w3_ref, b3_ref, wg_ref, h_ref):
    x = x_ref[...]
    l1 = jnp.dot(x, w1_ref[...], preferred_element_type=jnp.float32) + b1_ref[...]
    l1 = jnp.where(l1 > 0, l1, 0.01 * l1)
    l3 = jnp.dot(l1, w3_ref[...], preferred_element_type=jnp.float32) + b3_ref[...]
    l3 = jnp.where(l3 > 0, l3, 0.01 * l3)
    h_ref[...] = jnp.dot(l3, wg_ref[...], preferred_element_type=jnp.float32)


def _fused_mlp(x, W1, b1, W3, b3, Wg):
    grid = (N // ROWB,)
    return pl.pallas_call(
        _mlp_block,
        grid=grid,
        in_specs=[
            pl.BlockSpec((ROWB, LATENT), lambda i: (i, 0)),
            pl.BlockSpec((LATENT, LATENT), lambda i: (0, 0)),
            pl.BlockSpec((1, LATENT), lambda i: (0, 0)),
            pl.BlockSpec((LATENT, DIM_HALF), lambda i: (0, 0)),
            pl.BlockSpec((1, DIM_HALF), lambda i: (0, 0)),
            pl.BlockSpec((DIM_HALF, DIM_IN), lambda i: (0, 0)),
        ],
        out_specs=pl.BlockSpec((ROWB, DIM_IN), lambda i: (i, 0)),
        out_shape=jax.ShapeDtypeStruct((N, DIM_IN), jnp.float32),
    )(x, W1, b1.reshape(1, LATENT), W3, b3.reshape(1, DIM_HALF), Wg)


def _scale_block(h_ref, degp_ref, hs_ref, dinv_ref):
    deg = degp_ref[0] + degp_ref[1] + 1.0
    dinv = lax.rsqrt(deg)
    dinv_ref[...] = dinv
    hs_ref[...] = h_ref[...] * dinv


def _scale_rows(h, degp):
    grid = (N // ROWB,)
    return pl.pallas_call(
        _scale_block,
        grid=grid,
        in_specs=[
            pl.BlockSpec((ROWB, DIM_IN), lambda i: (i, 0)),
            pl.BlockSpec((NC, ROWB, 1), lambda i: (0, i, 0)),
        ],
        out_specs=[
            pl.BlockSpec((ROWB, DIM_IN), lambda i: (i, 0)),
            pl.BlockSpec((ROWB, 1), lambda i: (i, 0)),
        ],
        out_shape=[
            jax.ShapeDtypeStruct((N, DIM_IN), jnp.float32),
            jax.ShapeDtypeStruct((N, 1), jnp.float32),
        ],
    )(h, degp)


def _epilogue_block(acc_ref, hs_ref, dinv_ref, bg_ref, out_ref):
    v = (acc_ref[...] + hs_ref[...]) * dinv_ref[...] + bg_ref[...]
    out_ref[...] = jax.nn.sigmoid(v)


def _epilogue(acc, hs, dinv, bg):
    grid = (N // ROWB,)
    return pl.pallas_call(
        _epilogue_block,
        grid=grid,
        in_specs=[
            pl.BlockSpec((ROWB, DIM_IN), lambda i: (i, 0)),
            pl.BlockSpec((ROWB, DIM_IN), lambda i: (i, 0)),
            pl.BlockSpec((ROWB, 1), lambda i: (i, 0)),
            pl.BlockSpec((1, DIM_IN), lambda i: (0, 0)),
        ],
        out_specs=pl.BlockSpec((ROWB, DIM_IN), lambda i: (i, 0)),
        out_shape=jax.ShapeDtypeStruct((N, DIM_IN), jnp.float32),
    )(acc, hs, dinv, bg.reshape(1, DIM_IN))


def kernel(x, edge_index, batch, W1, b1, W3, b3, Wg, bg):
    del batch
    src = edge_index[0].astype(jnp.int32)
    dst = edge_index[1].astype(jnp.int32)
    degp = _degree_partials(dst).reshape(NC, HPAD)
    degp3 = degp[:, :N].reshape(NC, N, 1)
    h = _fused_mlp(x, W1, b1, W3, b3, Wg)
    hs, dinv = _scale_rows(h, degp3)
    zero = jnp.zeros((ZROWS, DIM_IN), jnp.float32)
    acc = _scatter_rows(hs, src, dst, zero)
    return _epilogue(acc, hs, dinv, bg)
